# baseline (speedup 1.0000x reference)
import jax
import jax.numpy as jnp
from jax.experimental import pallas as pl
from jax.experimental.pallas import tpu as pltpu

_BM = 200
_NBUF = 4


def _rgcn_body(adj_ref, x_hbm, w_ref, b_ref, o_ref, buf, xv, ostage, acc,
               sems, xsem, osems):
    n = x_hbm.shape[0]
    nrel = adj_ref.shape[0]
    nslab = nrel * (n // _BM)
    nblk = n // _BM

    def _copy(s, slot):
        r = jax.lax.rem(s, nrel)
        m = jax.lax.div(s, nrel)
        return pltpu.make_async_copy(
            adj_ref.at[r, pl.ds(pl.multiple_of(m * _BM, 8), _BM), :],
            buf.at[slot],
            sems.at[slot],
        )

    xcopy = pltpu.make_async_copy(x_hbm, xv, xsem)
    xcopy.start()
    for s0 in range(_NBUF):
        _copy(jnp.int32(s0), jnp.int32(s0)).start()
    xcopy.wait()

    def _ocopy(m, oslot):
        return pltpu.make_async_copy(
            ostage.at[oslot],
            o_ref.at[pl.ds(pl.multiple_of(m * _BM, 8), _BM), :],
            osems.at[oslot],
        )

    def _step(s, carry):
        slot = jax.lax.rem(s, _NBUF)
        r = jax.lax.rem(s, nrel)
        m = jax.lax.div(s, nrel)
        _copy(s, slot).wait()
        msg = jax.lax.dot(buf[slot], xv[...],
                          preferred_element_type=jnp.float32)
        part = jax.lax.dot(msg, w_ref[r], preferred_element_type=jnp.float32)

        @pl.when(r == 0)
        def _first():
            acc[...] = part

        @pl.when(r == nrel - 1)
        def _last():
            oslot = jax.lax.rem(m, 2)

            @pl.when(m >= 2)
            def _drain():
                _ocopy(m - 2, oslot).wait()

            ostage[oslot] = jnp.maximum(acc[...] + part + b_ref[...], 0.0)
            _ocopy(m, oslot).start()

        @pl.when(s + _NBUF < nslab)
        def _refill():
            _copy(s + _NBUF, slot).start()

        return carry

    jax.lax.fori_loop(0, nslab, _step, 0)

    for tail_m in (nblk - 2, nblk - 1):
        if tail_m >= 0:
            _ocopy(jnp.int32(tail_m), jnp.int32(tail_m % 2)).wait()


def kernel(node_features, adj_list, weight, bias):
    n, in_dim = node_features.shape
    r = adj_list.shape[0]
    out_dim = weight.shape[-1]

    b2 = bias.reshape(1, out_dim)

    return pl.pallas_call(
        _rgcn_body,
        in_specs=[
            pl.BlockSpec(memory_space=pl.ANY),
            pl.BlockSpec(memory_space=pl.ANY),
            pl.BlockSpec(memory_space=pltpu.VMEM),
            pl.BlockSpec(memory_space=pltpu.VMEM),
        ],
        out_specs=pl.BlockSpec(memory_space=pl.ANY),
        out_shape=jax.ShapeDtypeStruct((n, out_dim), jnp.float32),
        scratch_shapes=[
            pltpu.VMEM((_NBUF, _BM, n), jnp.float32),
            pltpu.VMEM((n, in_dim), jnp.float32),
            pltpu.VMEM((2, _BM, out_dim), jnp.float32),
            pltpu.VMEM((_BM, out_dim), jnp.float32),
            pltpu.SemaphoreType.DMA((_NBUF,)),
            pltpu.SemaphoreType.DMA,
            pltpu.SemaphoreType.DMA((2,)),
        ],
    )(adj_list, node_features, weight, b2)


# speedup vs baseline: 1.0294x; 1.0294x over previous
import jax
import jax.numpy as jnp
from jax.experimental import pallas as pl
from jax.experimental.pallas import tpu as pltpu

_BM = 200
_NBUF = 4


def _rgcn_body(adj_ref, x_ref, w_ref, b_ref, o_ref, buf, acc, sems):
    n = x_ref.shape[0]
    nrel = adj_ref.shape[0]
    nslab = nrel * (n // _BM)

    def _copy(s, slot):
        r = jax.lax.rem(s, nrel)
        m = jax.lax.div(s, nrel)
        return pltpu.make_async_copy(
            adj_ref.at[r, pl.ds(pl.multiple_of(m * _BM, 8), _BM), :],
            buf.at[slot],
            sems.at[slot],
        )

    for s0 in range(_NBUF):
        _copy(jnp.int32(s0), jnp.int32(s0)).start()

    def _step(s, carry):
        slot = jax.lax.rem(s, _NBUF)
        r = jax.lax.rem(s, nrel)
        m = jax.lax.div(s, nrel)
        _copy(s, slot).wait()
        msg = jax.lax.dot(buf[slot], x_ref[...],
                          preferred_element_type=jnp.float32)
        part = jax.lax.dot(msg, w_ref[r], preferred_element_type=jnp.float32)

        @pl.when(r == 0)
        def _first():
            acc[...] = part

        @pl.when(r == nrel - 1)
        def _last():
            row = pl.multiple_of(m * _BM, 8)
            o_ref[pl.ds(row, _BM), :] = jnp.maximum(
                acc[...] + part + b_ref[...], 0.0)

        @pl.when(s + _NBUF < nslab)
        def _refill():
            _copy(s + _NBUF, slot).start()

        return carry

    jax.lax.fori_loop(0, nslab, _step, 0)


def kernel(node_features, adj_list, weight, bias):
    n, in_dim = node_features.shape
    r = adj_list.shape[0]
    out_dim = weight.shape[-1]

    b2 = bias.reshape(1, out_dim)

    return pl.pallas_call(
        _rgcn_body,
        in_specs=[
            pl.BlockSpec(memory_space=pl.ANY),
            pl.BlockSpec(memory_space=pltpu.VMEM),
            pl.BlockSpec(memory_space=pltpu.VMEM),
            pl.BlockSpec(memory_space=pltpu.VMEM),
        ],
        out_specs=pl.BlockSpec(memory_space=pltpu.VMEM),
        out_shape=jax.ShapeDtypeStruct((n, out_dim), jnp.float32),
        scratch_shapes=[
            pltpu.VMEM((_NBUF, _BM, n), jnp.float32),
            pltpu.VMEM((_BM, out_dim), jnp.float32),
            pltpu.SemaphoreType.DMA((_NBUF,)),
        ],
    )(adj_list, node_features, weight, b2)
